# Initial kernel scaffold; baseline (speedup 1.0000x reference)
import jax
import jax.numpy as jnp
from jax.experimental import pallas as pl


def kernel(float_features, idlist_indices, idscore_indices, idscore_weights, emb_tables, w_emb_tables, W_dense, b_dense, W_over, b_over):
    raise NotImplementedError("write your pallas kernel here")


# speedup vs baseline: 3.5173x; 3.5173x over previous
import jax
import jax.numpy as jnp
from jax import lax
from jax.experimental import pallas as pl
from jax.experimental.pallas import tpu as pltpu
from jax.experimental.pallas import tpu_sc as plsc

_NT = 26
_NWT = 2
_NTT = _NT + _NWT
_VOCAB = 100000
_DIM = 16
_B = 4096
_L = 20
_NF = 10
_NC = 2
_NS = 16
_NW = _NC * _NS
_SAMP = _B // _NW
_CH = 64
_NCH = _SAMP * _L // _CH
_GRP = 8


def _sc_pool(idl, ids, wbro, emb, wemb, out, idx_v, rows_v, wrow_v, acc_v, sem):
    wid = lax.axis_index("s") * _NC + lax.axis_index("c")
    b0 = wid * _SAMP

    def gather_all(tbl):
        def grp(g, c):
            hs = []
            for k in range(_GRP):
                j = g * _GRP + k
                hs.append(pltpu.async_copy(
                    tbl.at[idx_v.at[j]],
                    rows_v.at[pl.ds(pl.multiple_of(j * _CH, _CH), _CH), :],
                    sem))
            for h in hs:
                h.wait()
            return c
        lax.fori_loop(0, _NCH // _GRP, grp, 0)

    def table_body(t, carry):
        pltpu.sync_copy(idl.at[t, pl.ds(wid * _NCH, _NCH), :], idx_v)
        gather_all(emb)

        def bag(s, c):
            r0 = s * _L
            v = rows_v[r0, :]
            for j in range(1, _L):
                v = v + rows_v[r0 + j, :]
            acc_v[s, :] = v
            return c

        lax.fori_loop(0, _SAMP, bag, carry)
        pltpu.sync_copy(acc_v, out.at[t, pl.ds(b0, _SAMP), :])
        return carry

    lax.fori_loop(0, _NT, table_body, 0)

    for w in range(_NWT):
        pltpu.sync_copy(ids.at[w, pl.ds(wid * _NCH, _NCH), :], idx_v)
        pltpu.sync_copy(wbro.at[w, pl.ds(b0 * _L, _SAMP * _L), :], wrow_v)
        gather_all(wemb)

        def wbag(s, c):
            r0 = s * _L
            v = rows_v[r0, :] * wrow_v[r0, :]
            for j in range(1, _L):
                v = v + rows_v[r0 + j, :] * wrow_v[r0 + j, :]
            acc_v[s, :] = v
            return c

        lax.fori_loop(0, _SAMP, wbag, 0)
        pltpu.sync_copy(acc_v, out.at[_NT + w, pl.ds(b0, _SAMP), :])


_sc_pool_call = pl.kernel(
    _sc_pool,
    out_type=jax.ShapeDtypeStruct((_NTT, _B, _DIM), jnp.float32),
    mesh=plsc.VectorSubcoreMesh(core_axis_name="c", subcore_axis_name="s"),
    compiler_params=pltpu.CompilerParams(use_tc_tiling_on_sc=False),
    scratch_types=[
        pltpu.VMEM((_NCH, _CH), jnp.int32),
        pltpu.VMEM((_SAMP * _L, _DIM), jnp.float32),
        pltpu.VMEM((_SAMP * _L, _DIM), jnp.float32),
        pltpu.VMEM((_SAMP, _DIM), jnp.float32),
        pltpu.SemaphoreType.DMA,
    ],
)


def _tc_body(ff, pooled, wd, bd, wo, bo, out):
    dense = jnp.dot(ff[...], wd[...], preferred_element_type=jnp.float32) + bd[...]
    r = jnp.dot(dense, wo[0:8, :], preferred_element_type=jnp.float32)
    for t in range(_NTT):
        r = r + jnp.dot(pooled[t], wo[8 + t * _DIM:8 + (t + 1) * _DIM, :],
                        preferred_element_type=jnp.float32)
    out[...] = r + bo[...]


_BLK = 512


def _tc_mix(ff, pooled, wd, bd, wo, bo):
    return pl.pallas_call(
        _tc_body,
        grid=(_B // _BLK,),
        in_specs=[
            pl.BlockSpec((_BLK, _NF), lambda i: (i, 0)),
            pl.BlockSpec((_NTT, _BLK, _DIM), lambda i: (0, i, 0)),
            pl.BlockSpec((_NF, 8), lambda i: (0, 0)),
            pl.BlockSpec((1, 8), lambda i: (0, 0)),
            pl.BlockSpec((8 + _NTT * _DIM, _DIM), lambda i: (0, 0)),
            pl.BlockSpec((1, _DIM), lambda i: (0, 0)),
        ],
        out_specs=pl.BlockSpec((_BLK, _DIM), lambda i: (i, 0)),
        out_shape=jax.ShapeDtypeStruct((_B, _DIM), jnp.float32),
    )(ff, pooled, wd, bd, wo, bo)


def kernel(float_features, idlist_indices, idscore_indices, idscore_weights,
           emb_tables, w_emb_tables, W_dense, b_dense, W_over, b_over):
    off = (jnp.arange(_NT, dtype=jnp.int32) * _VOCAB)[:, None, None]
    idl = idlist_indices.reshape(_NT, _B * _L // _CH, _CH) + off
    woff = (jnp.arange(_NWT, dtype=jnp.int32) * _VOCAB)[:, None, None]
    ids = idscore_indices.reshape(_NWT, _B * _L // _CH, _CH) + woff
    wbro = jnp.broadcast_to(idscore_weights.reshape(_NWT, _B * _L, 1),
                            (_NWT, _B * _L, _DIM))
    emb = emb_tables.reshape(_NT * _VOCAB, _DIM)
    wemb = w_emb_tables.reshape(_NWT * _VOCAB, _DIM)
    pooled = _sc_pool_call(idl, ids, wbro, emb, wemb)
    return _tc_mix(float_features, pooled, W_dense, b_dense.reshape(1, 8),
                   W_over, b_over.reshape(1, 16))


# speedup vs baseline: 3.9128x; 1.1125x over previous
import jax
import jax.numpy as jnp
from jax import lax
from jax.experimental import pallas as pl
from jax.experimental.pallas import tpu as pltpu
from jax.experimental.pallas import tpu_sc as plsc

_NT = 26
_NWT = 2
_NTT = _NT + _NWT
_VOCAB = 100000
_DIM = 16
_B = 4096
_L = 20
_NF = 10
_NC = 2
_NS = 16
_NW = _NC * _NS
_SAMP = _B // _NW
_CH = 128


def _sc_pool(idlT, ids, wbro, emb, wemb, out, idx_v, rows_v, wrow_v, pool_v, sem):
    wid = lax.axis_index("s") * _NC + lax.axis_index("c")
    b0 = wid * _SAMP

    def table_body(t, carry):
        pltpu.sync_copy(idlT.at[t, pl.ds(wid * _L, _L), :], idx_v)
        pltpu.async_copy(emb.at[idx_v.at[0]], pool_v, sem).wait()
        hs = [pltpu.async_copy(emb.at[idx_v.at[j]], pool_v, sem, add=True)
              for j in range(1, _L)]
        for h in hs:
            h.wait()
        pltpu.sync_copy(pool_v, out.at[t, pl.ds(b0, _SAMP), :])
        return carry

    lax.fori_loop(0, _NT, table_body, 0)

    for w in range(_NWT):
        pltpu.sync_copy(ids.at[w, pl.ds(wid * _L, _L), :], idx_v)
        pltpu.sync_copy(wbro.at[w, pl.ds(b0 * _L, _SAMP * _L), :], wrow_v)
        hs = [pltpu.async_copy(wemb.at[idx_v.at[j]],
                               rows_v.at[pl.ds(pl.multiple_of(j * _CH, _CH), _CH), :],
                               sem)
              for j in range(_L)]
        for h in hs:
            h.wait()

        def wbag(s, c):
            r0 = s * _L
            v = rows_v[r0, :] * wrow_v[r0, :]
            for j in range(1, _L):
                v = v + rows_v[r0 + j, :] * wrow_v[r0 + j, :]
            pool_v[s, :] = v
            return c

        lax.fori_loop(0, _SAMP, wbag, 0)
        pltpu.sync_copy(pool_v, out.at[_NT + w, pl.ds(b0, _SAMP), :])


_sc_pool_call = pl.kernel(
    _sc_pool,
    out_type=jax.ShapeDtypeStruct((_NTT, _B, _DIM), jnp.float32),
    mesh=plsc.VectorSubcoreMesh(core_axis_name="c", subcore_axis_name="s"),
    compiler_params=pltpu.CompilerParams(use_tc_tiling_on_sc=False),
    scratch_types=[
        pltpu.VMEM((_L, _CH), jnp.int32),
        pltpu.VMEM((_SAMP * _L, _DIM), jnp.float32),
        pltpu.VMEM((_SAMP * _L, _DIM), jnp.float32),
        pltpu.VMEM((_SAMP, _DIM), jnp.float32),
        pltpu.SemaphoreType.DMA,
    ],
)


def _tc_body(ff, pooled, wd, bd, wo, bo, out):
    dense = jnp.dot(ff[...], wd[...], preferred_element_type=jnp.float32) + bd[...]
    r = jnp.dot(dense, wo[0:8, :], preferred_element_type=jnp.float32)
    for t in range(_NTT):
        r = r + jnp.dot(pooled[t], wo[8 + t * _DIM:8 + (t + 1) * _DIM, :],
                        preferred_element_type=jnp.float32)
    out[...] = r + bo[...]


_BLK = 512


def _tc_mix(ff, pooled, wd, bd, wo, bo):
    return pl.pallas_call(
        _tc_body,
        grid=(_B // _BLK,),
        in_specs=[
            pl.BlockSpec((_BLK, _NF), lambda i: (i, 0)),
            pl.BlockSpec((_NTT, _BLK, _DIM), lambda i: (0, i, 0)),
            pl.BlockSpec((_NF, 8), lambda i: (0, 0)),
            pl.BlockSpec((1, 8), lambda i: (0, 0)),
            pl.BlockSpec((8 + _NTT * _DIM, _DIM), lambda i: (0, 0)),
            pl.BlockSpec((1, _DIM), lambda i: (0, 0)),
        ],
        out_specs=pl.BlockSpec((_BLK, _DIM), lambda i: (i, 0)),
        out_shape=jax.ShapeDtypeStruct((_B, _DIM), jnp.float32),
    )(ff, pooled, wd, bd, wo, bo)


def kernel(float_features, idlist_indices, idscore_indices, idscore_weights,
           emb_tables, w_emb_tables, W_dense, b_dense, W_over, b_over):
    off = (jnp.arange(_NT, dtype=jnp.int32) * _VOCAB)[:, None, None, None]
    idlT = (idlist_indices.reshape(_NT, _NW, _SAMP, _L) + off
            ).transpose(0, 1, 3, 2).reshape(_NT, _NW * _L, _SAMP)
    woff = (jnp.arange(_NWT, dtype=jnp.int32) * _VOCAB)[:, None, None]
    ids = idscore_indices.reshape(_NWT, _B * _L // _CH, _CH) + woff
    wbro = jnp.broadcast_to(idscore_weights.reshape(_NWT, _B * _L, 1),
                            (_NWT, _B * _L, _DIM))
    emb = emb_tables.reshape(_NT * _VOCAB, _DIM)
    wemb = w_emb_tables.reshape(_NWT * _VOCAB, _DIM)
    pooled = _sc_pool_call(idlT, ids, wbro, emb, wemb)
    return _tc_mix(float_features, pooled, W_dense, b_dense.reshape(1, 8),
                   W_over, b_over.reshape(1, 16))


# speedup vs baseline: 3.9291x; 1.0042x over previous
import jax
import jax.numpy as jnp
from jax import lax
from jax.experimental import pallas as pl
from jax.experimental.pallas import tpu as pltpu
from jax.experimental.pallas import tpu_sc as plsc

_NT = 26
_NWT = 2
_NTT = _NT + _NWT
_VOCAB = 100000
_DIM = 16
_B = 4096
_L = 20
_NF = 10
_NC = 2
_NS = 16
_NW = _NC * _NS
_SAMP = _B // _NW
_CH = 128


def _prep_body(idl, ids, idlT, idsT):
    toff = lax.broadcasted_iota(jnp.int32, (_NT, 1, 1), 0) * _VOCAB
    idlT[...] = jnp.transpose(idl[...], (0, 2, 1)) + toff
    woff = lax.broadcasted_iota(jnp.int32, (_NWT, 1, 1), 0) * _VOCAB
    idsT[...] = jnp.transpose(ids[...], (0, 2, 1)) + woff


_PBLK = 512


def _prep(idl, ids):
    return pl.pallas_call(
        _prep_body,
        grid=(_B // _PBLK,),
        in_specs=[
            pl.BlockSpec((_NT, _PBLK, _L), lambda i: (0, i, 0)),
            pl.BlockSpec((_NWT, _PBLK, _L), lambda i: (0, i, 0)),
        ],
        out_specs=[
            pl.BlockSpec((_NT, _L, _PBLK), lambda i: (0, 0, i)),
            pl.BlockSpec((_NWT, _L, _PBLK), lambda i: (0, 0, i)),
        ],
        out_shape=[
            jax.ShapeDtypeStruct((_NT, _L, _B), jnp.int32),
            jax.ShapeDtypeStruct((_NWT, _L, _B), jnp.int32),
        ],
    )(idl, ids)


def _sc_pool(idlT, idsT, wbro, emb, wemb, out, idx_v, rows_v, wrow_v, pool_v, sem):
    wid = lax.axis_index("s") * _NC + lax.axis_index("c")
    b0 = wid * _SAMP

    def table_body(t, carry):
        pltpu.sync_copy(idlT.at[t, :, pl.ds(b0, _SAMP)], idx_v)
        pltpu.async_copy(emb.at[idx_v.at[0]], pool_v, sem).wait()
        hs = [pltpu.async_copy(emb.at[idx_v.at[j]], pool_v, sem, add=True)
              for j in range(1, _L)]
        for h in hs:
            h.wait()
        pltpu.sync_copy(pool_v, out.at[t, pl.ds(b0, _SAMP), :])
        return carry

    lax.fori_loop(0, _NT, table_body, 0)

    for w in range(_NWT):
        pltpu.sync_copy(idsT.at[w, :, pl.ds(b0, _SAMP)], idx_v)
        pltpu.sync_copy(wbro.at[w, pl.ds(b0 * _L, _SAMP * _L), :], wrow_v)
        hs = [pltpu.async_copy(wemb.at[idx_v.at[j]],
                               rows_v.at[pl.ds(pl.multiple_of(j * _CH, _CH), _CH), :],
                               sem)
              for j in range(_L)]
        for h in hs:
            h.wait()

        def wbag(s, c):
            v = rows_v[s, :] * wrow_v[s * _L, :]
            for j in range(1, _L):
                v = v + rows_v[j * _CH + s, :] * wrow_v[s * _L + j, :]
            pool_v[s, :] = v
            return c

        lax.fori_loop(0, _SAMP, wbag, 0)
        pltpu.sync_copy(pool_v, out.at[_NT + w, pl.ds(b0, _SAMP), :])


_sc_pool_call = pl.kernel(
    _sc_pool,
    out_type=jax.ShapeDtypeStruct((_NTT, _B, _DIM), jnp.float32),
    mesh=plsc.VectorSubcoreMesh(core_axis_name="c", subcore_axis_name="s"),
    compiler_params=pltpu.CompilerParams(use_tc_tiling_on_sc=False),
    scratch_types=[
        pltpu.VMEM((_L, _CH), jnp.int32),
        pltpu.VMEM((_SAMP * _L, _DIM), jnp.float32),
        pltpu.VMEM((_SAMP * _L, _DIM), jnp.float32),
        pltpu.VMEM((_SAMP, _DIM), jnp.float32),
        pltpu.SemaphoreType.DMA,
    ],
)


def _tc_body(ff, pooled, wd, bd, wo, bo, out):
    dense = jnp.dot(ff[...], wd[...], preferred_element_type=jnp.float32) + bd[...]
    r = jnp.dot(dense, wo[0:8, :], preferred_element_type=jnp.float32)
    for t in range(_NTT):
        r = r + jnp.dot(pooled[t], wo[8 + t * _DIM:8 + (t + 1) * _DIM, :],
                        preferred_element_type=jnp.float32)
    out[...] = r + bo[...]


_BLK = 512


def _tc_mix(ff, pooled, wd, bd, wo, bo):
    return pl.pallas_call(
        _tc_body,
        grid=(_B // _BLK,),
        in_specs=[
            pl.BlockSpec((_BLK, _NF), lambda i: (i, 0)),
            pl.BlockSpec((_NTT, _BLK, _DIM), lambda i: (0, i, 0)),
            pl.BlockSpec((_NF, 8), lambda i: (0, 0)),
            pl.BlockSpec((1, 8), lambda i: (0, 0)),
            pl.BlockSpec((8 + _NTT * _DIM, _DIM), lambda i: (0, 0)),
            pl.BlockSpec((1, _DIM), lambda i: (0, 0)),
        ],
        out_specs=pl.BlockSpec((_BLK, _DIM), lambda i: (i, 0)),
        out_shape=jax.ShapeDtypeStruct((_B, _DIM), jnp.float32),
    )(ff, pooled, wd, bd, wo, bo)


def kernel(float_features, idlist_indices, idscore_indices, idscore_weights,
           emb_tables, w_emb_tables, W_dense, b_dense, W_over, b_over):
    idlT, idsT = _prep(idlist_indices, idscore_indices)
    wbro = jnp.broadcast_to(idscore_weights.reshape(_NWT, _B * _L, 1),
                            (_NWT, _B * _L, _DIM))
    emb = emb_tables.reshape(_NT * _VOCAB, _DIM)
    wemb = w_emb_tables.reshape(_NWT * _VOCAB, _DIM)
    pooled = _sc_pool_call(idlT, idsT, wbro, emb, wemb)
    return _tc_mix(float_features, pooled, W_dense, b_dense.reshape(1, 8),
                   W_over, b_over.reshape(1, 16))


# speedup vs baseline: 4.1994x; 1.0688x over previous
import jax
import jax.numpy as jnp
from jax import lax
from jax.experimental import pallas as pl
from jax.experimental.pallas import tpu as pltpu
from jax.experimental.pallas import tpu_sc as plsc

_NT = 26
_NWT = 2
_NTT = _NT + _NWT
_VOCAB = 100000
_DIM = 16
_B = 4096
_L = 20
_NF = 10
_NC = 2
_NS = 16
_NW = _NC * _NS
_SAMP = _B // _NW
_CH = 128
_PD = _NTT * _DIM


def _sc_pool(idlT, idsT, wts, emb, wemb, out, idx_v, rows_v, wts_v, pool_v, sem):
    wid = lax.axis_index("s") * _NC + lax.axis_index("c")
    b0 = wid * _SAMP

    def table_body(t, carry):
        pltpu.sync_copy(idlT.at[t, :, pl.ds(b0, _SAMP)], idx_v)
        pltpu.async_copy(emb.at[idx_v.at[0]], pool_v, sem).wait()
        hs = [pltpu.async_copy(emb.at[idx_v.at[j]], pool_v, sem, add=True)
              for j in range(1, _L)]
        for h in hs:
            h.wait()
        toff = pl.multiple_of(t * _DIM, _DIM)
        pltpu.sync_copy(pool_v, out.at[pl.ds(b0, _SAMP), pl.ds(toff, _DIM)])
        return carry

    lax.fori_loop(0, _NT, table_body, 0)

    dn = lax.GatherDimensionNumbers(offset_dims=(), collapsed_slice_dims=(0,),
                                    start_index_map=(0,))
    for w in range(_NWT):
        pltpu.sync_copy(idsT.at[w, :, pl.ds(b0, _SAMP)], idx_v)
        pltpu.sync_copy(wts.at[w, pl.ds(b0 * _L, _SAMP * _L)], wts_v)
        hs = [pltpu.async_copy(wemb.at[idx_v.at[j]],
                               rows_v.at[pl.ds(pl.multiple_of(j * _CH, _CH), _CH), :],
                               sem)
              for j in range(_L)]
        for h in hs:
            h.wait()

        def wbag(s, c):
            acc = None
            for j in range(_L):
                r = s * _L + j
                q = r // 16
                lane = jnp.full((16, 1), r - q * 16, jnp.int32)
                wvec = wts_v[pl.ds(pl.multiple_of(q * 16, 16), 16)]
                wb = lax.gather(wvec, lane, dn, (1,),
                                mode=lax.GatherScatterMode.PROMISE_IN_BOUNDS)
                rv = rows_v[j * _CH + s, :] * wb
                acc = rv if acc is None else acc + rv
            pool_v[s, :] = acc
            return c

        lax.fori_loop(0, _SAMP, wbag, 0)
        pltpu.sync_copy(pool_v,
                        out.at[pl.ds(b0, _SAMP), pl.ds((_NT + w) * _DIM, _DIM)])


_sc_pool_call = pl.kernel(
    _sc_pool,
    out_type=jax.ShapeDtypeStruct((_B, _PD), jnp.float32),
    mesh=plsc.VectorSubcoreMesh(core_axis_name="c", subcore_axis_name="s"),
    compiler_params=pltpu.CompilerParams(use_tc_tiling_on_sc=False),
    scratch_types=[
        pltpu.VMEM((_L, _CH), jnp.int32),
        pltpu.VMEM((_SAMP * _L, _DIM), jnp.float32),
        pltpu.VMEM((_SAMP * _L,), jnp.float32),
        pltpu.VMEM((_SAMP, _DIM), jnp.float32),
        pltpu.SemaphoreType.DMA,
    ],
)


def _tc_body(ff, pooled, wd, bd, wo, bo, out):
    dense = jnp.dot(ff[...], wd[...], preferred_element_type=jnp.float32) + bd[...]
    r = jnp.dot(dense, wo[0:8, :], preferred_element_type=jnp.float32)
    r = r + jnp.dot(pooled[...], wo[8:, :], preferred_element_type=jnp.float32)
    out[...] = r + bo[...]


_BLK = 512


def _tc_mix(ff, pooled, wd, bd, wo, bo):
    return pl.pallas_call(
        _tc_body,
        grid=(_B // _BLK,),
        in_specs=[
            pl.BlockSpec((_BLK, _NF), lambda i: (i, 0)),
            pl.BlockSpec((_BLK, _PD), lambda i: (i, 0)),
            pl.BlockSpec((_NF, 8), lambda i: (0, 0)),
            pl.BlockSpec((1, 8), lambda i: (0, 0)),
            pl.BlockSpec((8 + _PD, _DIM), lambda i: (0, 0)),
            pl.BlockSpec((1, _DIM), lambda i: (0, 0)),
        ],
        out_specs=pl.BlockSpec((_BLK, _DIM), lambda i: (i, 0)),
        out_shape=jax.ShapeDtypeStruct((_B, _DIM), jnp.float32),
    )(ff, pooled, wd, bd, wo, bo)


def kernel(float_features, idlist_indices, idscore_indices, idscore_weights,
           emb_tables, w_emb_tables, W_dense, b_dense, W_over, b_over):
    off = (jnp.arange(_NT, dtype=jnp.int32) * _VOCAB)[:, None, None]
    idlT = jnp.transpose(idlist_indices, (0, 2, 1)) + off
    woff = (jnp.arange(_NWT, dtype=jnp.int32) * _VOCAB)[:, None, None]
    idsT = jnp.transpose(idscore_indices, (0, 2, 1)) + woff
    wts = idscore_weights.reshape(_NWT, _B * _L)
    emb = emb_tables.reshape(_NT * _VOCAB, _DIM)
    wemb = w_emb_tables.reshape(_NWT * _VOCAB, _DIM)
    pooled = _sc_pool_call(idlT, idsT, wts, emb, wemb)
    return _tc_mix(float_features, pooled, W_dense, b_dense.reshape(1, 8),
                   W_over, b_over.reshape(1, 16))


# speedup vs baseline: 8.9231x; 2.1249x over previous
import jax
import jax.numpy as jnp
from jax import lax
from jax.experimental import pallas as pl
from jax.experimental.pallas import tpu as pltpu
from jax.experimental.pallas import tpu_sc as plsc

_NT = 26
_NWT = 2
_NTT = _NT + _NWT
_VOCAB = 100000
_VS = 100096
_DIM = 16
_B = 4096
_L = 20
_NF = 10
_NC = 2
_NS = 16
_NW = _NC * _NS
_SAMP = _B // _NW
_CH = 128
_PD = _NTT * _DIM

_VC = 5888
_RPB = _VC * _DIM // 128
_RPT = _VS * _DIM // 128


def _fmt_body(src, dst):
    x = src[0]
    z = jnp.concatenate([x[:, i * _RPB:(i + 1) * _RPB] for i in range(8)],
                        axis=0)
    dst[...] = jnp.transpose(z, (1, 0))


def _fmt(tblT, nt):
    return pl.pallas_call(
        _fmt_body,
        grid=(nt, _VS // _VC),
        in_specs=[pl.BlockSpec((1, _DIM, _VC), lambda t, v: (t, 0, v))],
        out_specs=pl.BlockSpec((_RPB, 128), lambda t, v: (t * (_VS // _VC) + v, 0)),
        out_shape=jax.ShapeDtypeStruct((nt * _RPT, 128), jnp.float32),
    )(tblT)


def _sc_pool(idlT, idsT, wts, emb, wemb, out, idx_v, rows_v, wts_v, pool_v, sem):
    wid = lax.axis_index("s") * _NC + lax.axis_index("c")
    b0 = wid * _SAMP

    def table_body(t, carry):
        pltpu.sync_copy(idlT.at[t, :, pl.ds(b0, _SAMP)], idx_v)
        pltpu.async_copy(emb.at[idx_v.at[0]], pool_v, sem).wait()
        hs = [pltpu.async_copy(emb.at[idx_v.at[j]], pool_v, sem, add=True)
              for j in range(1, _L)]
        for h in hs:
            h.wait()
        toff = pl.multiple_of(t * _DIM, _DIM)
        pltpu.sync_copy(pool_v, out.at[pl.ds(b0, _SAMP), pl.ds(toff, _DIM)])
        return carry

    lax.fori_loop(0, _NT, table_body, 0)

    dn = lax.GatherDimensionNumbers(offset_dims=(), collapsed_slice_dims=(0,),
                                    start_index_map=(0,))
    for w in range(_NWT):
        pltpu.sync_copy(idsT.at[w, :, pl.ds(b0, _SAMP)], idx_v)
        pltpu.sync_copy(wts.at[w, pl.ds(b0 * _L, _SAMP * _L)], wts_v)
        hs = [pltpu.async_copy(wemb.at[idx_v.at[j]],
                               rows_v.at[pl.ds(pl.multiple_of(j * _CH, _CH), _CH), :],
                               sem)
              for j in range(_L)]
        for h in hs:
            h.wait()

        def wbag(s, c):
            acc = None
            for j in range(_L):
                r = s * _L + j
                q = r // 16
                lane = jnp.full((16, 1), r - q * 16, jnp.int32)
                wvec = wts_v[pl.ds(pl.multiple_of(q * 16, 16), 16)]
                wb = lax.gather(wvec, lane, dn, (1,),
                                mode=lax.GatherScatterMode.PROMISE_IN_BOUNDS)
                rv = rows_v[j * _CH + s, :] * wb
                acc = rv if acc is None else acc + rv
            pool_v[s, :] = acc
            return c

        lax.fori_loop(0, _SAMP, wbag, 0)
        pltpu.sync_copy(pool_v,
                        out.at[pl.ds(b0, _SAMP), pl.ds((_NT + w) * _DIM, _DIM)])


_sc_pool_call = pl.kernel(
    _sc_pool,
    out_type=jax.ShapeDtypeStruct((_B, _PD), jnp.float32),
    mesh=plsc.VectorSubcoreMesh(core_axis_name="c", subcore_axis_name="s"),
    compiler_params=pltpu.CompilerParams(use_tc_tiling_on_sc=False),
    scratch_types=[
        pltpu.VMEM((_L, _CH), jnp.int32),
        pltpu.VMEM((_SAMP * _L, _DIM), jnp.float32),
        pltpu.VMEM((_SAMP * _L,), jnp.float32),
        pltpu.VMEM((_SAMP, _DIM), jnp.float32),
        pltpu.SemaphoreType.DMA,
    ],
)


def _tc_body(ff, pooled, wd, bd, wo, bo, out):
    dense = jnp.dot(ff[...], wd[...], preferred_element_type=jnp.float32) + bd[...]
    r = jnp.dot(dense, wo[0:8, :], preferred_element_type=jnp.float32)
    r = r + jnp.dot(pooled[...], wo[8:, :], preferred_element_type=jnp.float32)
    out[...] = r + bo[...]


_BLK = 512


def _tc_mix(ff, pooled, wd, bd, wo, bo):
    return pl.pallas_call(
        _tc_body,
        grid=(_B // _BLK,),
        in_specs=[
            pl.BlockSpec((_BLK, _NF), lambda i: (i, 0)),
            pl.BlockSpec((_BLK, _PD), lambda i: (i, 0)),
            pl.BlockSpec((_NF, 8), lambda i: (0, 0)),
            pl.BlockSpec((1, 8), lambda i: (0, 0)),
            pl.BlockSpec((8 + _PD, _DIM), lambda i: (0, 0)),
            pl.BlockSpec((1, _DIM), lambda i: (0, 0)),
        ],
        out_specs=pl.BlockSpec((_BLK, _DIM), lambda i: (i, 0)),
        out_shape=jax.ShapeDtypeStruct((_B, _DIM), jnp.float32),
    )(ff, pooled, wd, bd, wo, bo)


def _remap(idxT, nt):
    tb = (jnp.arange(nt, dtype=jnp.int32) * (_VS // _VC))[:, None, None]
    vb = idxT // _VC
    vr = idxT - vb * _VC
    return ((tb + vb) * _RPB + vr % _RPB) * 8 + vr // _RPB


def kernel(float_features, idlist_indices, idscore_indices, idscore_weights,
           emb_tables, w_emb_tables, W_dense, b_dense, W_over, b_over):
    idlT = _remap(jnp.transpose(idlist_indices, (0, 2, 1)), _NT)
    idsT = _remap(jnp.transpose(idscore_indices, (0, 2, 1)), _NWT)
    wts = idscore_weights.reshape(_NWT, _B * _L)
    emb = _fmt(jnp.transpose(emb_tables, (0, 2, 1)), _NT
               ).reshape(_NT * _VS, _DIM)
    wemb = _fmt(jnp.transpose(w_emb_tables, (0, 2, 1)), _NWT
                ).reshape(_NWT * _VS, _DIM)
    pooled = _sc_pool_call(idlT, idsT, wts, emb, wemb)
    return _tc_mix(float_features, pooled, W_dense, b_dense.reshape(1, 8),
                   W_over, b_over.reshape(1, 16))
